# Initial kernel scaffold; baseline (speedup 1.0000x reference)
import jax
import jax.numpy as jnp
from jax.experimental import pallas as pl


def kernel(text, table, fc_w, fc_b):
    raise NotImplementedError("write your pallas kernel here")


# speedup vs baseline: 2.8620x; 2.8620x over previous
import functools

import jax
import jax.numpy as jnp
from jax import lax
from jax.experimental import pallas as pl
from jax.experimental.pallas import tpu as pltpu
from jax.experimental.pallas import tpu_sc as plsc

VOCAB = 1000000
EMBED_DIM = 64
BATCH = 16384
HIST = 200
NUM_CLASSES = 2

DP = 16
PACK = 8
VP = VOCAB // PACK
NC, NS = 2, 16
NW = NC * NS
BAGS_PER_W = BATCH // NW
G = 8
ROWS_PER_CHUNK = 2 * G
CHUNKS_PER_W = BAGS_PER_W // G
TC_BLK = 1000


def _project_table(table512, big_w, b_row):
    def body(t_ref, w_ref, b_ref, o_ref):
        o_ref[...] = (
            jnp.dot(t_ref[...], w_ref[...], preferred_element_type=jnp.float32)
            + b_ref[...]
        )

    return pl.pallas_call(
        body,
        grid=(VP // TC_BLK,),
        in_specs=[
            pl.BlockSpec((TC_BLK, PACK * EMBED_DIM), lambda i: (i, 0)),
            pl.BlockSpec((PACK * EMBED_DIM, PACK * DP), lambda i: (0, 0)),
            pl.BlockSpec((1, PACK * DP), lambda i: (0, 0)),
        ],
        out_specs=pl.BlockSpec((TC_BLK, PACK * DP), lambda i: (i, 0)),
        out_shape=jax.ShapeDtypeStruct((VP, PACK * DP), jnp.float32),
    )(table512, big_w, b_row)


def _bag_sum(proj, idx3):
    mesh = plsc.VectorSubcoreMesh(core_axis_name="c", subcore_axis_name="s")

    @functools.partial(
        pl.kernel,
        mesh=mesh,
        out_type=jax.ShapeDtypeStruct((BATCH, DP), jnp.float32),
        compiler_params=pltpu.CompilerParams(use_tc_tiling_on_sc=False),
        name="bag_sum",
        scratch_types=[
            pltpu.VMEM((ROWS_PER_CHUNK, 100), jnp.int32),
            pltpu.VMEM((G * HIST, DP), jnp.float32),
            pltpu.VMEM((G, DP), jnp.float32),
            pltpu.SemaphoreType.DMA,
        ],
    )
    def body(proj_hbm, idx_hbm, out_hbm, idx_v, rows_v, out_v, sem):
        wid = lax.axis_index("s") * NC + lax.axis_index("c")

        def chunk_body(c, carry):
            gchunk = wid * CHUNKS_PER_W + c
            pltpu.sync_copy(idx_hbm.at[gchunk], idx_v)
            copies = [
                pltpu.async_copy(
                    proj_hbm.at[idx_v.at[j]],
                    rows_v.at[pl.ds(j * 100, 100)],
                    sem,
                )
                for j in range(ROWS_PER_CHUNK)
            ]
            for cp in copies:
                cp.wait()
            for g in range(G):
                base = g * HIST

                def acc_body(i, accs, base=base):
                    r = base + i * 4
                    return tuple(accs[k] + rows_v[r + k, :] for k in range(4))

                z = jnp.zeros((DP,), jnp.float32)
                a = lax.fori_loop(0, HIST // 4, acc_body, (z, z, z, z))
                out_v[g, :] = (a[0] + a[1]) + (a[2] + a[3])
            pltpu.sync_copy(out_v, out_hbm.at[pl.ds(gchunk * G, G)])
            return carry

        lax.fori_loop(0, CHUNKS_PER_W, chunk_body, 0)

    return body(proj, idx3)


def kernel(text, table, fc_w, fc_b):
    inv_l = 1.0 / HIST
    w_pad = jnp.zeros((EMBED_DIM, DP), jnp.float32).at[:, :NUM_CLASSES].set(
        fc_w.T * inv_l
    )
    big_w = jnp.kron(jnp.eye(PACK, dtype=jnp.float32), w_pad)
    b_pad = jnp.zeros((DP,), jnp.float32).at[:NUM_CLASSES].set(fc_b * inv_l)
    b_row = jnp.tile(b_pad, PACK)[None, :]
    table512 = table.reshape(VP, PACK * EMBED_DIM)
    proj128 = _project_table(table512, big_w, b_row)
    proj = proj128.reshape(VOCAB, DP)
    idx3 = text.astype(jnp.int32).reshape(NW * CHUNKS_PER_W, ROWS_PER_CHUNK, 100)
    out16 = _bag_sum(proj, idx3)
    return out16[:, :NUM_CLASSES]


# speedup vs baseline: 4.0151x; 1.4029x over previous
import functools

import jax
import jax.numpy as jnp
from jax import lax
from jax.experimental import pallas as pl
from jax.experimental.pallas import tpu as pltpu
from jax.experimental.pallas import tpu_sc as plsc

VOCAB = 1000000
EMBED_DIM = 64
BATCH = 16384
HIST = 200
NUM_CLASSES = 2

DP = 16
PACK = 8
VP = VOCAB // PACK
NC, NS = 2, 16
NW = NC * NS
BAGS_PER_W = BATCH // NW
G = 4
ROWS_PER_CHUNK = 2 * G
CHUNKS_PER_W = BAGS_PER_W // G
TC_BLK = 1000


def _project_table(table, big_w, b_row):
    def body(*refs):
        t_refs, w_ref, b_ref, o_ref = refs[:PACK], refs[PACK], refs[PACK + 1], refs[PACK + 2]
        x = jnp.concatenate([t[...] for t in t_refs], axis=1)
        o_ref[...] = (
            jnp.dot(x, w_ref[...], preferred_element_type=jnp.float32)
            + b_ref[...]
        )

    nblk = VP // TC_BLK

    def _tmap(k):
        return lambda i, k=k: (k * nblk + i, 0)

    return pl.pallas_call(
        body,
        grid=(nblk,),
        in_specs=[
            pl.BlockSpec((TC_BLK, EMBED_DIM), _tmap(k)) for k in range(PACK)
        ] + [
            pl.BlockSpec((PACK * EMBED_DIM, PACK * DP), lambda i: (0, 0)),
            pl.BlockSpec((1, PACK * DP), lambda i: (0, 0)),
        ],
        out_specs=pl.BlockSpec((TC_BLK, PACK * DP), lambda i: (i, 0)),
        out_shape=jax.ShapeDtypeStruct((VP, PACK * DP), jnp.float32),
    )(*([table] * PACK), big_w, b_row)


def _bag_sum(proj, idx3):
    mesh = plsc.VectorSubcoreMesh(core_axis_name="c", subcore_axis_name="s")

    rows_n = ROWS_PER_CHUNK * 100

    @functools.partial(
        pl.kernel,
        mesh=mesh,
        out_type=jax.ShapeDtypeStruct((BATCH, DP), jnp.float32),
        compiler_params=pltpu.CompilerParams(use_tc_tiling_on_sc=False),
        name="bag_sum",
        scratch_types=[
            pltpu.VMEM((2, ROWS_PER_CHUNK, 100), jnp.int32),
            pltpu.VMEM((2, rows_n, DP), jnp.float32),
            pltpu.VMEM((G, DP), jnp.float32),
            pltpu.SemaphoreType.DMA,
            pltpu.SemaphoreType.DMA,
            pltpu.SemaphoreType.DMA,
            pltpu.SemaphoreType.DMA,
        ],
    )
    def body(proj_hbm, idx_hbm, out_hbm, idx_v, rows_v, out_v,
             sem0, sem1, isem0, isem1):
        wid = lax.axis_index("s") * NC + lax.axis_index("c")
        sems = (sem0, sem1)
        isems = (isem0, isem1)
        last = CHUNKS_PER_W - 1

        def idx_start(c, p):
            pltpu.async_copy(
                idx_hbm.at[wid * CHUNKS_PER_W + c], idx_v.at[p], isems[p]
            )

        def idx_drain(p):
            pltpu.make_async_copy(
                idx_hbm.at[0], idx_v.at[p], isems[p]
            ).wait()

        def fire(p):
            for j in range(ROWS_PER_CHUNK):
                pltpu.async_copy(
                    proj_hbm.at[idx_v.at[p].at[j]],
                    rows_v.at[p].at[pl.ds(j * 100, 100)],
                    sems[p],
                )

        def drain(p):
            pltpu.make_async_copy(
                proj_hbm.at[pl.ds(0, rows_n)], rows_v.at[p], sems[p]
            ).wait()

        pltpu.sync_copy(idx_hbm.at[wid * CHUNKS_PER_W], idx_v.at[0])
        fire(0)
        idx_start(1, 1)

        def pair_body(k, carry):
            for p in (0, 1):
                c = 2 * k + p
                idx_drain(1 - p)
                fire(1 - p)
                drain(p)
                idx_start(jnp.minimum(c + 2, last), p)
                rv = rows_v.at[p]
                for g in range(G):
                    base = g * HIST

                    def acc_body(i, accs, base=base, rv=rv):
                        r = base + i * 4
                        return tuple(accs[j] + rv[r + j, :] for j in range(4))

                    z = jnp.zeros((DP,), jnp.float32)
                    a = lax.fori_loop(0, HIST // 4, acc_body, (z, z, z, z))
                    out_v[g, :] = (a[0] + a[1]) + (a[2] + a[3])
                pltpu.sync_copy(
                    out_v,
                    out_hbm.at[pl.ds((wid * CHUNKS_PER_W + c) * G, G)],
                )
            return carry

        lax.fori_loop(0, CHUNKS_PER_W // 2, pair_body, 0)
        drain(0)
        idx_drain(1)

    return body(proj, idx3)


def kernel(text, table, fc_w, fc_b):
    inv_l = 1.0 / HIST
    w_pad = jnp.zeros((EMBED_DIM, DP), jnp.float32).at[:, :NUM_CLASSES].set(
        fc_w.T * inv_l
    )
    big_w = jnp.kron(jnp.eye(PACK, dtype=jnp.float32), w_pad)
    b_pad = jnp.zeros((DP,), jnp.float32).at[:NUM_CLASSES].set(fc_b * inv_l)
    b_row = jnp.tile(b_pad, PACK)[None, :]
    proj128 = _project_table(table, big_w, b_row)
    proj = proj128.reshape(VOCAB, DP)
    t32 = text.astype(jnp.int32)
    tt = (t32 % VP) * 8 + t32 // VP
    idx3 = tt.reshape(NW * CHUNKS_PER_W, ROWS_PER_CHUNK, 100)
    out16 = _bag_sum(proj, idx3)
    return out16[:, :NUM_CLASSES]


# speedup vs baseline: 4.1576x; 1.0355x over previous
import functools

import jax
import jax.numpy as jnp
from jax import lax
from jax.experimental import pallas as pl
from jax.experimental.pallas import tpu as pltpu
from jax.experimental.pallas import tpu_sc as plsc

VOCAB = 1000000
EMBED_DIM = 64
BATCH = 16384
HIST = 200
NUM_CLASSES = 2

DP = 16
PACK = 8
VP = VOCAB // PACK
NC, NS = 2, 16
NW = NC * NS
BAGS_PER_W = BATCH // NW
G = 4
ROWS_PER_CHUNK = 2 * G
CHUNKS_PER_W = BAGS_PER_W // G
TC_BLK = 1000


def _project_table(table, big_w, b_row):
    def body(*refs):
        t_refs, w_ref, b_ref, o_ref = refs[:PACK], refs[PACK], refs[PACK + 1], refs[PACK + 2]
        x = jnp.concatenate([t[...] for t in t_refs], axis=1)
        o_ref[...] = (
            jnp.dot(x, w_ref[...], preferred_element_type=jnp.float32)
            + b_ref[...]
        )

    nblk = VP // TC_BLK

    def _tmap(k):
        return lambda i, k=k: (k * nblk + i, 0)

    return pl.pallas_call(
        body,
        grid=(nblk,),
        in_specs=[
            pl.BlockSpec((TC_BLK, EMBED_DIM), _tmap(k)) for k in range(PACK)
        ] + [
            pl.BlockSpec((PACK * EMBED_DIM, PACK * DP), lambda i: (0, 0)),
            pl.BlockSpec((1, PACK * DP), lambda i: (0, 0)),
        ],
        out_specs=pl.BlockSpec((TC_BLK, PACK * DP), lambda i: (i, 0)),
        out_shape=jax.ShapeDtypeStruct((VP, PACK * DP), jnp.float32),
    )(*([table] * PACK), big_w, b_row)


def _bag_sum(proj, idx3):
    mesh = plsc.VectorSubcoreMesh(core_axis_name="c", subcore_axis_name="s")

    rows_n = ROWS_PER_CHUNK * 100

    @functools.partial(
        pl.kernel,
        mesh=mesh,
        out_type=jax.ShapeDtypeStruct((BATCH, DP), jnp.float32),
        compiler_params=pltpu.CompilerParams(use_tc_tiling_on_sc=False),
        name="bag_sum",
        scratch_types=[
            pltpu.VMEM((2, ROWS_PER_CHUNK, 100), jnp.int32),
            pltpu.VMEM((2, rows_n, DP), jnp.float32),
            pltpu.VMEM((G, DP), jnp.float32),
            pltpu.SemaphoreType.DMA,
            pltpu.SemaphoreType.DMA,
            pltpu.SemaphoreType.DMA,
            pltpu.SemaphoreType.DMA,
        ],
    )
    def body(proj_hbm, idx_hbm, out_hbm, idx_v, rows_v, out_v,
             sem0, sem1, isem0, isem1):
        wid = lax.axis_index("s") * NC + lax.axis_index("c")
        sems = (sem0, sem1)
        isems = (isem0, isem1)
        last = CHUNKS_PER_W - 1

        def idx_start(c, p):
            pltpu.async_copy(
                idx_hbm.at[wid * CHUNKS_PER_W + c], idx_v.at[p], isems[p]
            )

        def idx_drain(p):
            pltpu.make_async_copy(
                idx_hbm.at[0], idx_v.at[p], isems[p]
            ).wait()

        def fire(p):
            for j in range(ROWS_PER_CHUNK):
                pltpu.async_copy(
                    proj_hbm.at[idx_v.at[p].at[j]],
                    rows_v.at[p].at[pl.ds(j * 100, 100)],
                    sems[p],
                )

        def drain(p):
            pltpu.make_async_copy(
                proj_hbm.at[pl.ds(0, rows_n)], rows_v.at[p], sems[p]
            ).wait()

        pltpu.sync_copy(idx_hbm.at[wid * CHUNKS_PER_W], idx_v.at[0])
        fire(0)
        idx_start(1, 1)

        def pair_body(k, carry):
            for p in (0, 1):
                c = 2 * k + p
                idx_drain(1 - p)
                fire(1 - p)
                drain(p)
                idx_start(jnp.minimum(c + 2, last), p)
                rv = rows_v.at[p]
                for g in range(G):
                    base = g * HIST

                    def acc_body(i, accs, base=base, rv=rv):
                        r = base + i * 4
                        return tuple(accs[j] + rv[r + j, :] for j in range(4))

                    z = jnp.zeros((DP,), jnp.float32)
                    a = lax.fori_loop(0, HIST // 4, acc_body, (z, z, z, z))
                    out_v[g, :] = (a[0] + a[1]) + (a[2] + a[3])
                pltpu.sync_copy(
                    out_v,
                    out_hbm.at[pl.ds((wid * CHUNKS_PER_W + c) * G, G)],
                )
            return carry

        lax.fori_loop(0, CHUNKS_PER_W // 2, pair_body, 0)
        drain(0)
        idx_drain(1)

    return body(proj, idx3)


def kernel(text, table, fc_w, fc_b):
    inv_l = 1.0 / HIST
    w_pad = jnp.zeros((EMBED_DIM, DP), jnp.float32).at[:, :NUM_CLASSES].set(
        fc_w.T * inv_l
    )
    big_w = jnp.kron(jnp.eye(PACK, dtype=jnp.float32), w_pad)
    b_pad = jnp.zeros((DP,), jnp.float32).at[:NUM_CLASSES].set(fc_b * inv_l)
    b_row = jnp.tile(b_pad, PACK)[None, :]
    proj128 = _project_table(
        table.astype(jnp.bfloat16), big_w.astype(jnp.bfloat16), b_row
    )
    proj = proj128.reshape(VOCAB, DP)
    t32 = text.astype(jnp.int32)
    tt = (t32 % VP) * 8 + t32 // VP
    idx3 = tt.reshape(NW * CHUNKS_PER_W, ROWS_PER_CHUNK, 100)
    out16 = _bag_sum(proj, idx3)
    return out16[:, :NUM_CLASSES]


# speedup vs baseline: 4.4749x; 1.0763x over previous
import functools

import jax
import jax.numpy as jnp
from jax import lax
from jax.experimental import pallas as pl
from jax.experimental.pallas import tpu as pltpu
from jax.experimental.pallas import tpu_sc as plsc

VOCAB = 1000000
EMBED_DIM = 64
BATCH = 16384
HIST = 200
NUM_CLASSES = 2

DP = 16
PACK = 8
VP = VOCAB // PACK
NC, NS = 2, 16
NW = NC * NS
BAGS_PER_W = BATCH // NW
G = 4
ROWS_PER_CHUNK = 2 * G
CHUNKS_PER_W = BAGS_PER_W // G
TC_BLK = 5000


def _project_table(table, big_w, b_row):
    def body(*refs):
        t_refs, w_ref, b_ref, o_ref = refs[:PACK], refs[PACK], refs[PACK + 1], refs[PACK + 2]
        x = jnp.concatenate([t[...] for t in t_refs], axis=1)
        o_ref[...] = (
            jnp.dot(x, w_ref[...], preferred_element_type=jnp.float32)
            + b_ref[...]
        )

    nblk = VP // TC_BLK

    def _tmap(k):
        return lambda i, k=k: (k * nblk + i, 0)

    return pl.pallas_call(
        body,
        grid=(nblk,),
        in_specs=[
            pl.BlockSpec((TC_BLK, EMBED_DIM), _tmap(k)) for k in range(PACK)
        ] + [
            pl.BlockSpec((PACK * EMBED_DIM, PACK * DP), lambda i: (0, 0)),
            pl.BlockSpec((1, PACK * DP), lambda i: (0, 0)),
        ],
        out_specs=pl.BlockSpec((TC_BLK, PACK * DP), lambda i: (i, 0)),
        out_shape=jax.ShapeDtypeStruct((VP, PACK * DP), jnp.float32),
    )(*([table] * PACK), big_w, b_row)


def _bag_sum(proj, idx3):
    mesh = plsc.VectorSubcoreMesh(core_axis_name="c", subcore_axis_name="s")

    rows_n = ROWS_PER_CHUNK * 100

    @functools.partial(
        pl.kernel,
        mesh=mesh,
        out_type=jax.ShapeDtypeStruct((BATCH, DP), jnp.float32),
        compiler_params=pltpu.CompilerParams(use_tc_tiling_on_sc=False),
        name="bag_sum",
        scratch_types=[
            pltpu.VMEM((2, ROWS_PER_CHUNK, 100), jnp.int32),
            pltpu.VMEM((2, rows_n, DP), jnp.float32),
            pltpu.VMEM((G, DP), jnp.float32),
            pltpu.SemaphoreType.DMA,
            pltpu.SemaphoreType.DMA,
            pltpu.SemaphoreType.DMA,
            pltpu.SemaphoreType.DMA,
        ],
    )
    def body(proj_hbm, idx_hbm, out_hbm, idx_v, rows_v, out_v,
             sem0, sem1, isem0, isem1):
        wid = lax.axis_index("s") * NC + lax.axis_index("c")
        sems = (sem0, sem1)
        isems = (isem0, isem1)
        last = CHUNKS_PER_W - 1

        def idx_start(c, p):
            pltpu.async_copy(
                idx_hbm.at[wid * CHUNKS_PER_W + c], idx_v.at[p], isems[p]
            )

        def idx_drain(p):
            pltpu.make_async_copy(
                idx_hbm.at[0], idx_v.at[p], isems[p]
            ).wait()

        def fire(p):
            for j in range(ROWS_PER_CHUNK):
                pltpu.async_copy(
                    proj_hbm.at[idx_v.at[p].at[j]],
                    rows_v.at[p].at[pl.ds(j * 100, 100)],
                    sems[p],
                )

        def drain(p):
            pltpu.make_async_copy(
                proj_hbm.at[pl.ds(0, rows_n)], rows_v.at[p], sems[p]
            ).wait()

        pltpu.sync_copy(idx_hbm.at[wid * CHUNKS_PER_W], idx_v.at[0])
        fire(0)
        idx_start(1, 1)

        def pair_body(k, carry):
            for p in (0, 1):
                c = 2 * k + p
                idx_drain(1 - p)
                fire(1 - p)
                drain(p)
                idx_start(jnp.minimum(c + 2, last), p)
                rv = rows_v.at[p]
                for g in range(G):
                    base = g * HIST

                    def acc_body(i, accs, base=base, rv=rv):
                        r = base + i * 4
                        return tuple(accs[j] + rv[r + j, :] for j in range(4))

                    z = jnp.zeros((DP,), jnp.float32)
                    a = lax.fori_loop(0, HIST // 4, acc_body, (z, z, z, z))
                    out_v[g, :] = (a[0] + a[1]) + (a[2] + a[3])
                pltpu.sync_copy(
                    out_v,
                    out_hbm.at[pl.ds((wid * CHUNKS_PER_W + c) * G, G)],
                )
            return carry

        lax.fori_loop(0, CHUNKS_PER_W // 2, pair_body, 0)
        drain(0)
        idx_drain(1)

    return body(proj, idx3)


def kernel(text, table, fc_w, fc_b):
    inv_l = 1.0 / HIST
    w_pad = jnp.zeros((EMBED_DIM, DP), jnp.float32).at[:, :NUM_CLASSES].set(
        fc_w.T * inv_l
    )
    big_w = jnp.kron(jnp.eye(PACK, dtype=jnp.float32), w_pad)
    b_pad = jnp.zeros((DP,), jnp.float32).at[:NUM_CLASSES].set(fc_b * inv_l)
    b_row = jnp.tile(b_pad, PACK)[None, :]
    proj128 = _project_table(
        table.astype(jnp.bfloat16), big_w.astype(jnp.bfloat16), b_row
    )
    proj = proj128.reshape(VOCAB, DP)
    t32 = text.astype(jnp.int32)
    tt = (t32 % VP) * 8 + t32 // VP
    idx3 = tt.reshape(NW * CHUNKS_PER_W, ROWS_PER_CHUNK, 100)
    out16 = _bag_sum(proj, idx3)
    return out16[:, :NUM_CLASSES]


# speedup vs baseline: 4.5043x; 1.0066x over previous
import functools

import jax
import jax.numpy as jnp
from jax import lax
from jax.experimental import pallas as pl
from jax.experimental.pallas import tpu as pltpu
from jax.experimental.pallas import tpu_sc as plsc

VOCAB = 1000000
EMBED_DIM = 64
BATCH = 16384
HIST = 200
NUM_CLASSES = 2

DP = 16
PACK = 8
VP = VOCAB // PACK
NC, NS = 2, 16
NW = NC * NS
BAGS_PER_W = BATCH // NW
G = 4
ROWS_PER_CHUNK = 2 * G
CHUNKS_PER_W = BAGS_PER_W // G
TC_BLK = 5000


def _project_table(table, big_w, b_row):
    def body(*refs):
        t_refs, w_ref, b_ref, o_ref = refs[:PACK], refs[PACK], refs[PACK + 1], refs[PACK + 2]
        x = jnp.concatenate([t[...] for t in t_refs], axis=1)
        o_ref[...] = (
            jnp.dot(x, w_ref[...], preferred_element_type=jnp.float32)
            + b_ref[...]
        )

    nblk = VP // TC_BLK

    def _tmap(k):
        return lambda i, k=k: (k * nblk + i, 0)

    return pl.pallas_call(
        body,
        grid=(nblk,),
        in_specs=[
            pl.BlockSpec((TC_BLK, EMBED_DIM), _tmap(k)) for k in range(PACK)
        ] + [
            pl.BlockSpec((PACK * EMBED_DIM, PACK * DP), lambda i: (0, 0)),
            pl.BlockSpec((1, PACK * DP), lambda i: (0, 0)),
        ],
        out_specs=pl.BlockSpec((TC_BLK, PACK * DP), lambda i: (i, 0)),
        out_shape=jax.ShapeDtypeStruct((VP, PACK * DP), jnp.float32),
    )(*([table] * PACK), big_w, b_row)


def _bag_sum(proj, idx3):
    mesh = plsc.VectorSubcoreMesh(core_axis_name="c", subcore_axis_name="s")

    rows_n = ROWS_PER_CHUNK * 100

    @functools.partial(
        pl.kernel,
        mesh=mesh,
        out_type=jax.ShapeDtypeStruct((BATCH, DP), jnp.float32),
        compiler_params=pltpu.CompilerParams(use_tc_tiling_on_sc=False),
        name="bag_sum",
        scratch_types=[
            pltpu.VMEM((2, ROWS_PER_CHUNK, 100), jnp.int32),
            pltpu.VMEM((2, rows_n, DP), jnp.float32),
            pltpu.VMEM((G, DP), jnp.float32),
            pltpu.SemaphoreType.DMA,
            pltpu.SemaphoreType.DMA,
            pltpu.SemaphoreType.DMA,
            pltpu.SemaphoreType.DMA,
        ],
    )
    def body(proj_hbm, idx_hbm, out_hbm, idx_v, rows_v, out_v,
             sem0, sem1, isem0, isem1):
        wid = lax.axis_index("s") * NC + lax.axis_index("c")
        sems = (sem0, sem1)
        isems = (isem0, isem1)
        last = CHUNKS_PER_W - 1

        def idx_start(c, p):
            pltpu.async_copy(
                idx_hbm.at[wid * CHUNKS_PER_W + c], idx_v.at[p], isems[p]
            )

        def idx_drain(p):
            pltpu.make_async_copy(
                idx_hbm.at[0], idx_v.at[p], isems[p]
            ).wait()

        def fire(p):
            for j in range(ROWS_PER_CHUNK):
                pltpu.async_copy(
                    proj_hbm.at[idx_v.at[p].at[j]],
                    rows_v.at[p].at[pl.ds(j * 100, 100)],
                    sems[p],
                )

        def drain(p):
            pltpu.make_async_copy(
                proj_hbm.at[pl.ds(0, rows_n)], rows_v.at[p], sems[p]
            ).wait()

        pltpu.sync_copy(idx_hbm.at[wid * CHUNKS_PER_W], idx_v.at[0])
        fire(0)
        idx_start(1, 1)

        def pair_body(k, carry):
            for p in (0, 1):
                c = 2 * k + p
                idx_drain(1 - p)
                fire(1 - p)
                drain(p)
                idx_start(jnp.minimum(c + 2, last), p)
                rv = rows_v.at[p]
                for g in range(G):
                    base = g * HIST

                    def acc_body(i, accs, base=base, rv=rv):
                        r = base + i * 8
                        return tuple(accs[j] + rv[r + j, :] for j in range(8))

                    z = jnp.zeros((DP,), jnp.float32)
                    a = lax.fori_loop(0, HIST // 8, acc_body, (z,) * 8)
                    out_v[g, :] = (
                        ((a[0] + a[1]) + (a[2] + a[3]))
                        + ((a[4] + a[5]) + (a[6] + a[7]))
                    )
                pltpu.sync_copy(
                    out_v,
                    out_hbm.at[pl.ds((wid * CHUNKS_PER_W + c) * G, G)],
                )
            return carry

        lax.fori_loop(0, CHUNKS_PER_W // 2, pair_body, 0)
        drain(0)
        idx_drain(1)

    return body(proj, idx3)


def kernel(text, table, fc_w, fc_b):
    inv_l = 1.0 / HIST
    w_pad = jnp.zeros((EMBED_DIM, DP), jnp.float32).at[:, :NUM_CLASSES].set(
        fc_w.T * inv_l
    )
    big_w = jnp.kron(jnp.eye(PACK, dtype=jnp.float32), w_pad)
    b_pad = jnp.zeros((DP,), jnp.float32).at[:NUM_CLASSES].set(fc_b * inv_l)
    b_row = jnp.tile(b_pad, PACK)[None, :]
    proj128 = _project_table(
        table.astype(jnp.bfloat16), big_w.astype(jnp.bfloat16), b_row
    )
    proj = proj128.reshape(VOCAB, DP)
    t32 = text.astype(jnp.int32)
    tt = (t32 % VP) * 8 + t32 // VP
    idx3 = tt.reshape(NW * CHUNKS_PER_W, ROWS_PER_CHUNK, 100)
    out16 = _bag_sum(proj, idx3)
    return out16[:, :NUM_CLASSES]
